# baseline (speedup 1.0000x reference)
import jax
import jax.numpy as jnp
from jax.experimental import pallas as pl
from jax.experimental.pallas import tpu as pltpu

_B = 32
_A = 3
_C = 80
_CH = _C + 5
_GH = 64
_GW = 64
_NPOS = _GH * _GW
_STRIDE = 8.0
_AW = (116.0, 156.0, 373.0)
_AH = (90.0, 198.0, 326.0)


def _decode_body(x_ref, o_ref):
    a = pl.program_id(1)
    v = x_ref[0, 0]
    sig = jax.nn.sigmoid(v)

    top = v[0:8, :]
    ex = jnp.exp(top)
    row = jax.lax.broadcasted_iota(jnp.int32, top.shape, 0)
    col = jax.lax.broadcasted_iota(jnp.int32, top.shape, 1)
    gx = (col % _GW).astype(jnp.float32)

    aw = jnp.where(a == 0, _AW[0], jnp.where(a == 1, _AW[1], _AW[2]))
    ah = jnp.where(a == 0, _AH[0], jnp.where(a == 1, _AH[1], _AH[2]))

    sig_top = sig[0:8, :]
    top_out = jnp.where(
        row < 2,
        (sig_top + gx) * _STRIDE,
        jnp.where(row == 2, ex * aw, jnp.where(row == 3, ex * ah, sig_top)),
    )
    out = jnp.concatenate([top_out, sig[8:, :]], axis=0)
    o_ref[0] = out.T


def kernel(x):
    x4 = x.reshape(_B, _A, _CH, _NPOS)
    return pl.pallas_call(
        _decode_body,
        grid=(_B, _A),
        in_specs=[
            pl.BlockSpec((1, 1, _CH, _NPOS), lambda b, a: (b, a, 0, 0)),
        ],
        out_specs=pl.BlockSpec((1, _NPOS, _CH), lambda b, a: (b, a, 0)),
        out_shape=jax.ShapeDtypeStruct((_B, _A * _NPOS, _CH), jnp.float32),
    )(x4)


# speedup vs baseline: 1.0009x; 1.0009x over previous
import jax
import jax.numpy as jnp
from jax.experimental import pallas as pl
from jax.experimental.pallas import tpu as pltpu

_B = 32
_A = 3
_C = 80
_CH = _C + 5
_GH = 64
_GW = 64
_NPOS = _GH * _GW
_STRIDE = 8.0
_AW = (116.0, 156.0, 373.0)
_AH = (90.0, 198.0, 326.0)


def _decode_body(x_ref, o_ref):
    a = pl.program_id(1)
    v = x_ref[0, 0]
    sig = jnp.tanh(v * 0.5) * 0.5 + 0.5

    top = v[0:8, :]
    ex = jnp.exp(top)
    row = jax.lax.broadcasted_iota(jnp.int32, top.shape, 0)
    col = jax.lax.broadcasted_iota(jnp.int32, top.shape, 1)
    gx = (col % _GW).astype(jnp.float32)

    aw = jnp.where(a == 0, _AW[0], jnp.where(a == 1, _AW[1], _AW[2]))
    ah = jnp.where(a == 0, _AH[0], jnp.where(a == 1, _AH[1], _AH[2]))

    sig_top = sig[0:8, :]
    top_out = jnp.where(
        row < 2,
        (sig_top + gx) * _STRIDE,
        jnp.where(row == 2, ex * aw, jnp.where(row == 3, ex * ah, sig_top)),
    )
    out = jnp.concatenate([top_out, sig[8:, :]], axis=0)
    o_ref[0] = out.T


def kernel(x):
    x4 = x.reshape(_B, _A, _CH, _NPOS)
    return pl.pallas_call(
        _decode_body,
        grid=(_B, _A),
        in_specs=[
            pl.BlockSpec((1, 1, _CH, _NPOS), lambda b, a: (b, a, 0, 0)),
        ],
        out_specs=pl.BlockSpec((1, _NPOS, _CH), lambda b, a: (b, a, 0)),
        out_shape=jax.ShapeDtypeStruct((_B, _A * _NPOS, _CH), jnp.float32),
    )(x4)


# speedup vs baseline: 1.4670x; 1.4657x over previous
import jax
import jax.numpy as jnp
from jax.experimental import pallas as pl
from jax.experimental.pallas import tpu as pltpu

_B = 32
_A = 3
_C = 80
_CH = _C + 5
_GH = 64
_GW = 64
_NPOS = _GH * _GW
_STRIDE = 8.0
_AW = (116.0, 156.0, 373.0)
_AH = (90.0, 198.0, 326.0)


def _decode_body(x_ref, o_ref):
    a = pl.program_id(1)
    v = x_ref[0].reshape(_CH, _NPOS)
    sig = jnp.tanh(v * 0.5) * 0.5 + 0.5

    top = v[0:8, :]
    ex = jnp.exp(top)
    row = jax.lax.broadcasted_iota(jnp.int32, top.shape, 0)
    col = jax.lax.broadcasted_iota(jnp.int32, top.shape, 1)
    gx = (col % _GW).astype(jnp.float32)

    aw = jnp.where(a == 0, _AW[0], jnp.where(a == 1, _AW[1], _AW[2]))
    ah = jnp.where(a == 0, _AH[0], jnp.where(a == 1, _AH[1], _AH[2]))

    sig_top = sig[0:8, :]
    top_out = jnp.where(
        row < 2,
        (sig_top + gx) * _STRIDE,
        jnp.where(row == 2, ex * aw, jnp.where(row == 3, ex * ah, sig_top)),
    )
    out = jnp.concatenate([top_out, sig[8:, :]], axis=0)
    o_ref[0] = out.T


def kernel(x):
    return pl.pallas_call(
        _decode_body,
        grid=(_B, _A),
        in_specs=[
            pl.BlockSpec((1, _CH, _GH, _GW), lambda b, a: (b, a, 0, 0)),
        ],
        out_specs=pl.BlockSpec((1, _NPOS, _CH), lambda b, a: (b, a, 0)),
        out_shape=jax.ShapeDtypeStruct((_B, _A * _NPOS, _CH), jnp.float32),
    )(x)


# speedup vs baseline: 3.4604x; 2.3588x over previous
import jax
import jax.numpy as jnp
from jax.experimental import pallas as pl
from jax.experimental.pallas import tpu as pltpu

_B = 32
_A = 3
_C = 80
_CH = _C + 5
_NCH = _A * _CH
_GH = 64
_GW = 64
_NPOS = _GH * _GW
_STRIDE = 8.0
_AW = (116.0, 156.0, 373.0)
_AH = (90.0, 198.0, 326.0)


def _decode_body(x_ref, o_ref):
    v = x_ref[0]
    c = jax.lax.broadcasted_iota(jnp.int32, v.shape, 2)
    cm = c % _CH
    gx = (jax.lax.broadcasted_iota(jnp.int32, v.shape, 1) % _GW).astype(
        jnp.float32
    )

    sig = jnp.tanh(v * 0.5) * 0.5 + 0.5
    ex = jnp.exp(v)

    aw = jnp.where(c < _CH, _AW[0], jnp.where(c < 2 * _CH, _AW[1], _AW[2]))
    ah = jnp.where(c < _CH, _AH[0], jnp.where(c < 2 * _CH, _AH[1], _AH[2]))

    y = jnp.where(
        cm < 2,
        (sig + gx) * _STRIDE,
        jnp.where(cm == 2, ex * aw, jnp.where(cm == 3, ex * ah, sig)),
    )
    t = y.reshape(_NPOS, _NCH).T
    o_ref[:, 0, 0:32, :] = t[0:_CH].reshape(_CH, 32, 128)
    o_ref[:, 0, 32:64, :] = t[_CH : 2 * _CH].reshape(_CH, 32, 128)
    o_ref[:, 0, 64:96, :] = t[2 * _CH :].reshape(_CH, 32, 128)


def kernel(x):
    x3 = x.reshape(_B, _NCH, 32, 128).transpose(0, 2, 3, 1)
    res = pl.pallas_call(
        _decode_body,
        grid=(_B,),
        in_specs=[
            pl.BlockSpec((1, 32, 128, _NCH), lambda b: (b, 0, 0, 0)),
        ],
        out_specs=pl.BlockSpec(
            (_CH, 1, 96, 128), lambda b: (0, b, 0, 0)
        ),
        out_shape=jax.ShapeDtypeStruct((_CH, _B, 96, 128), jnp.float32),
    )(x3)
    return res.reshape(_CH, _B, _A * _NPOS).transpose(1, 2, 0)


# speedup vs baseline: 3.5896x; 1.0373x over previous
import jax
import jax.numpy as jnp
from jax.experimental import pallas as pl
from jax.experimental.pallas import tpu as pltpu

_B = 32
_A = 3
_C = 80
_CH = _C + 5
_NCH = _A * _CH
_GH = 64
_GW = 64
_NPOS = _GH * _GW
_STRIDE = 8.0
_AW = (116.0, 156.0, 373.0)
_AH = (90.0, 198.0, 326.0)


def _decode_body(x_ref, o_ref):
    for i in range(x_ref.shape[0]):
        _decode_one(x_ref, o_ref, i)


def _decode_one(x_ref, o_ref, i):
    v = x_ref[i]
    c = jax.lax.broadcasted_iota(jnp.int32, v.shape, 2)
    cm = c % _CH
    gx = (jax.lax.broadcasted_iota(jnp.int32, v.shape, 1) % _GW).astype(
        jnp.float32
    )

    sig = jnp.tanh(v * 0.5) * 0.5 + 0.5
    ex = jnp.exp(v)

    aw = jnp.where(c < _CH, _AW[0], jnp.where(c < 2 * _CH, _AW[1], _AW[2]))
    ah = jnp.where(c < _CH, _AH[0], jnp.where(c < 2 * _CH, _AH[1], _AH[2]))

    y = jnp.where(
        cm < 2,
        (sig + gx) * _STRIDE,
        jnp.where(cm == 2, ex * aw, jnp.where(cm == 3, ex * ah, sig)),
    )
    t = y.reshape(_NPOS, _NCH).T
    o_ref[:, i, 0:32, :] = t[0:_CH].reshape(_CH, 32, 128)
    o_ref[:, i, 32:64, :] = t[_CH : 2 * _CH].reshape(_CH, 32, 128)
    o_ref[:, i, 64:96, :] = t[2 * _CH :].reshape(_CH, 32, 128)


def kernel(x):
    x3 = x.reshape(_B, _NCH, 32, 128).transpose(0, 2, 3, 1)
    res = pl.pallas_call(
        _decode_body,
        grid=(_B // 2,),
        in_specs=[
            pl.BlockSpec((2, 32, 128, _NCH), lambda b: (b, 0, 0, 0)),
        ],
        out_specs=pl.BlockSpec(
            (_CH, 2, 96, 128), lambda b: (0, b, 0, 0)
        ),
        out_shape=jax.ShapeDtypeStruct((_CH, _B, 96, 128), jnp.float32),
    )(x3)
    return res.reshape(_CH, _B, _A * _NPOS).transpose(1, 2, 0)


# speedup vs baseline: 3.7343x; 1.0403x over previous
import jax
import jax.numpy as jnp
from jax.experimental import pallas as pl
from jax.experimental.pallas import tpu as pltpu

_B = 32
_A = 3
_C = 80
_CH = _C + 5
_NCH = _A * _CH
_GH = 64
_GW = 64
_NPOS = _GH * _GW
_STRIDE = 8.0
_AW = (116.0, 156.0, 373.0)
_AH = (90.0, 198.0, 326.0)


def _coeffs():
    c = jax.lax.broadcasted_iota(jnp.int32, (128, _NCH), 1)
    cm = c % _CH
    aw = jnp.where(c < _CH, _AW[0], jnp.where(c < 2 * _CH, _AW[1], _AW[2]))
    ah = jnp.where(c < _CH, _AH[0], jnp.where(c < 2 * _CH, _AH[1], _AH[2]))
    s_sig = jnp.where(cm < 2, _STRIDE, jnp.where(cm < 4, 0.0, 1.0))
    sa = s_sig * 0.5
    sb = jnp.where(cm == 2, aw, jnp.where(cm == 3, ah, 0.0))
    gx = (jax.lax.broadcasted_iota(jnp.int32, (128, _NCH), 0) % _GW).astype(
        jnp.float32
    )
    so = sa + jnp.where(cm < 2, gx * _STRIDE, 0.0)
    return sa, sb, so


def _decode_body(x_ref, o_ref):
    sa, sb, so = _coeffs()
    for i in range(x_ref.shape[0]):
        v = x_ref[i]
        y = jnp.tanh(v * 0.5) * sa + jnp.exp(v) * sb + so
        t = y.reshape(_NPOS, _NCH).T
        o_ref[:, i, 0:32, :] = t[0:_CH].reshape(_CH, 32, 128)
        o_ref[:, i, 32:64, :] = t[_CH : 2 * _CH].reshape(_CH, 32, 128)
        o_ref[:, i, 64:96, :] = t[2 * _CH :].reshape(_CH, 32, 128)


def kernel(x):
    x3 = x.reshape(_B, _NCH, 32, 128).transpose(0, 2, 3, 1)
    res = pl.pallas_call(
        _decode_body,
        grid=(_B // 2,),
        in_specs=[
            pl.BlockSpec((2, 32, 128, _NCH), lambda b: (b, 0, 0, 0)),
        ],
        out_specs=pl.BlockSpec(
            (_CH, 2, 96, 128), lambda b: (0, b, 0, 0)
        ),
        out_shape=jax.ShapeDtypeStruct((_CH, _B, 96, 128), jnp.float32),
    )(x3)
    return res.reshape(_CH, _B, _A * _NPOS).transpose(1, 2, 0)
